# baseline (speedup 1.0000x reference)
import functools

import jax
import jax.numpy as jnp
from jax import lax
from jax.experimental import pallas as pl
from jax.experimental.pallas import tpu as pltpu
from jax.experimental.pallas import tpu_sc as plsc

VOCAB = 1000
VPAD = 1024
LPR = VPAD // 128
T = 20
TPAD = 24
NC = 2
NS = 16
NW = NC * NS


def _make_gather(b):
    ktok = T // 2
    kr = ktok * LPR
    b_per_w = b // NW
    nchunk = 2 * b_per_w
    assert nchunk % 2 == 0 and kr <= 128
    rows_per_b = TPAD * LPR
    mesh = plsc.VectorSubcoreMesh(core_axis_name="c", subcore_axis_name="s")

    @functools.partial(
        pl.kernel,
        out_type=jax.ShapeDtypeStruct((b * rows_per_b, 128), jnp.float32),
        mesh=mesh,
        scratch_types=[
            pltpu.VMEM((b_per_w * T * LPR,), jnp.int32),
            pltpu.VMEM((2, kr, 128), jnp.float32),
            pltpu.SemaphoreType.DMA,
            pltpu.SemaphoreType.DMA,
        ],
        compiler_params=pltpu.CompilerParams(use_tc_tiling_on_sc=True),
    )
    def gather_kernel(tview_hbm, idx_hbm, out_hbm, idx_v, rows_v, sem0, sem1):
        wid = lax.axis_index("s") * NC + lax.axis_index("c")
        idx_base = wid * b_per_w * T * LPR
        out_base = wid * b_per_w * rows_per_b
        sems = (sem0, sem1)
        pltpu.sync_copy(idx_hbm.at[pl.ds(idx_base, b_per_w * T * LPR)], idx_v)

        def gather_dma(c, slot):
            return pltpu.make_async_copy(
                tview_hbm.at[idx_v.at[pl.ds(c * kr, kr)]],
                rows_v.at[slot],
                sems[slot],
            )

        def out_copy(c, slot):
            off = out_base + (c // 2) * rows_per_b + (c % 2) * kr
            pltpu.sync_copy(rows_v.at[slot], out_hbm.at[pl.ds(off, kr)])

        gather_dma(0, 0).start()

        def body(c2, carry):
            c = 2 * c2
            gather_dma(c + 1, 1).start()
            gather_dma(c, 0).wait()
            out_copy(c, 0)
            gather_dma(c + 2, 0).start()
            gather_dma(c + 1, 1).wait()
            out_copy(c + 1, 1)
            return carry

        lax.fori_loop(0, nchunk // 2 - 1, body, 0)
        c = nchunk - 2
        gather_dma(c + 1, 1).start()
        gather_dma(c, 0).wait()
        out_copy(c, 0)
        gather_dma(c + 1, 1).wait()
        out_copy(c + 1, 1)

    return gather_kernel


def _relayout_body(x_ref, o_ref):
    o_ref[...] = x_ref[:, :T, :VOCAB]


def _make_relayout(b, bb):
    return pl.pallas_call(
        _relayout_body,
        grid=(b // bb,),
        in_specs=[pl.BlockSpec((bb, TPAD, VPAD), lambda i: (i, 0, 0))],
        out_specs=pl.BlockSpec((bb, T, VOCAB), lambda i: (i, 0, 0)),
        out_shape=jax.ShapeDtypeStruct((b, T, VOCAB), jnp.float32),
    )


_gather = _make_gather(4096)
_relayout = _make_relayout(4096, 32)


@jax.jit
def kernel(idx, token_embedding_table):
    b, t = idx.shape
    flat = idx.reshape(b * t)
    idx8 = (flat[:, None] * LPR + jnp.arange(LPR, dtype=jnp.int32)).reshape(-1)
    table_p = jnp.pad(token_embedding_table, ((0, 0), (0, VPAD - VOCAB)))
    tview = table_p.reshape(VOCAB * LPR, 128)
    mid = _gather(tview, idx8)
    return _relayout(mid.reshape(b, TPAD, VPAD))


# speedup vs baseline: 2.2652x; 2.2652x over previous
import functools

import jax
import jax.numpy as jnp
from jax import lax
from jax.experimental import pallas as pl
from jax.experimental.pallas import tpu as pltpu
from jax.experimental.pallas import tpu_sc as plsc

VOCAB = 1000
VPAD = 1024
LPR = VPAD // 128
T = 20
B = 4096
NC = 2
NS = 16
NW = NC * NS


def _make_gather():
    ktok = 16
    kr = ktok * LPR
    b_per_w = B // NW
    cpt = b_per_w // ktok
    nchunk = T * cpt
    mesh = plsc.VectorSubcoreMesh(core_axis_name="c", subcore_axis_name="s")

    @functools.partial(
        pl.kernel,
        out_type=jax.ShapeDtypeStruct((B * T * LPR, 128), jnp.float32),
        mesh=mesh,
        scratch_types=[
            pltpu.VMEM((T * b_per_w * LPR,), jnp.int32),
            pltpu.VMEM((2, kr, 128), jnp.float32),
            pltpu.SemaphoreType.DMA,
            pltpu.SemaphoreType.DMA,
        ],
        compiler_params=pltpu.CompilerParams(use_tc_tiling_on_sc=True),
    )
    def gather_kernel(tview_hbm, idx_hbm, out_hbm, idx_v, rows_v, sem0, sem1):
        wid = lax.axis_index("s") * NC + lax.axis_index("c")
        sems = (sem0, sem1)
        for t in range(T):
            pltpu.sync_copy(
                idx_hbm.at[pl.ds(t * (B * LPR) + wid * (b_per_w * LPR),
                                 b_per_w * LPR)],
                idx_v.at[pl.ds(t * (b_per_w * LPR), b_per_w * LPR)],
            )

        def gather_dma(c, slot):
            return pltpu.make_async_copy(
                tview_hbm.at[idx_v.at[pl.ds(c * kr, kr)]],
                rows_v.at[slot],
                sems[slot],
            )

        def out_copy(c, slot):
            off = ((c // cpt) * (B * LPR) + wid * (b_per_w * LPR)
                   + (c % cpt) * kr)
            pltpu.sync_copy(rows_v.at[slot], out_hbm.at[pl.ds(off, kr)])

        gather_dma(0, 0).start()

        def body(c2, carry):
            c = 2 * c2
            gather_dma(c + 1, 1).start()
            gather_dma(c, 0).wait()
            out_copy(c, 0)
            gather_dma(c + 2, 0).start()
            gather_dma(c + 1, 1).wait()
            out_copy(c + 1, 1)
            return carry

        lax.fori_loop(0, nchunk // 2 - 1, body, 0)
        c = nchunk - 2
        gather_dma(c + 1, 1).start()
        gather_dma(c, 0).wait()
        out_copy(c, 0)
        gather_dma(c + 1, 1).wait()
        out_copy(c + 1, 1)

    return gather_kernel


BJ = 512


def _transpose_body(x_ref, o_ref):
    x = x_ref[...]
    xr = x.reshape(BJ, LPR, 128)
    y = jnp.transpose(xr, (1, 0, 2))
    z = jnp.transpose(y, (0, 2, 1))
    o_ref[0] = z.reshape(VPAD, BJ)[:VOCAB]


def _make_transpose():
    return pl.pallas_call(
        _transpose_body,
        grid=(T, B // BJ),
        in_specs=[pl.BlockSpec((BJ * LPR, 128), lambda t, j: (t * (B // BJ) + j, 0))],
        out_specs=pl.BlockSpec((1, VOCAB, BJ), lambda t, j: (t, 0, j)),
        out_shape=jax.ShapeDtypeStruct((T, VOCAB, B), jnp.float32),
    )


_gather = _make_gather()
_transpose = _make_transpose()


@jax.jit
def kernel(idx, token_embedding_table):
    idx_t = jnp.transpose(idx)
    idx8 = (idx_t.reshape(-1)[:, None] * LPR
            + jnp.arange(LPR, dtype=jnp.int32)).reshape(-1)
    table_p = jnp.pad(token_embedding_table, ((0, 0), (0, VPAD - VOCAB)))
    tview = table_p.reshape(VOCAB * LPR, 128)
    mid = _gather(tview, idx8)
    out_t = _transpose(mid)
    return jnp.transpose(out_t, (2, 0, 1))


# speedup vs baseline: 2.4140x; 1.0657x over previous
import functools

import jax
import jax.numpy as jnp
from jax import lax
from jax.experimental import pallas as pl
from jax.experimental.pallas import tpu as pltpu
from jax.experimental.pallas import tpu_sc as plsc

VOCAB = 1000
VPAD = 1024
LPR = VPAD // 128
T = 20
B = 4096
NG = 4
TG = T // NG
NC = 2
NS = 16
NW = NC * NS


def _make_gather(g):
    ktok = 16
    kr = ktok * LPR
    b_per_w = B // NW
    cpt = b_per_w // ktok
    nchunk = TG * cpt
    assert nchunk % 2 == 0 and nchunk >= 4
    g_off = g * TG * B * LPR
    mesh = plsc.VectorSubcoreMesh(core_axis_name="c", subcore_axis_name="s")

    @functools.partial(
        pl.kernel,
        out_type=jax.ShapeDtypeStruct((B * TG * LPR, 128), jnp.float32),
        mesh=mesh,
        scratch_types=[
            pltpu.VMEM((TG * b_per_w * LPR,), jnp.int32),
            pltpu.VMEM((2, kr, 128), jnp.float32),
            pltpu.SemaphoreType.DMA,
            pltpu.SemaphoreType.DMA,
        ],
        compiler_params=pltpu.CompilerParams(use_tc_tiling_on_sc=True),
    )
    def gather_kernel(tview_hbm, idx_hbm, out_hbm, idx_v, rows_v, sem0, sem1):
        wid = lax.axis_index("s") * NC + lax.axis_index("c")
        sems = (sem0, sem1)
        for t in range(TG):
            pltpu.sync_copy(
                idx_hbm.at[pl.ds(g_off + t * (B * LPR) + wid * (b_per_w * LPR),
                                 b_per_w * LPR)],
                idx_v.at[pl.ds(t * (b_per_w * LPR), b_per_w * LPR)],
            )

        def gather_dma(c, slot):
            return pltpu.make_async_copy(
                tview_hbm.at[idx_v.at[pl.ds(c * kr, kr)]],
                rows_v.at[slot],
                sems[slot],
            )

        def out_copy(c, slot):
            off = ((c // cpt) * (B * LPR) + wid * (b_per_w * LPR)
                   + (c % cpt) * kr)
            pltpu.sync_copy(rows_v.at[slot], out_hbm.at[pl.ds(off, kr)])

        gather_dma(0, 0).start()

        def body(c2, carry):
            c = 2 * c2
            gather_dma(c + 1, 1).start()
            gather_dma(c, 0).wait()
            out_copy(c, 0)
            gather_dma(c + 2, 0).start()
            gather_dma(c + 1, 1).wait()
            out_copy(c + 1, 1)
            return carry

        lax.fori_loop(0, nchunk // 2 - 1, body, 0)
        c = nchunk - 2
        gather_dma(c + 1, 1).start()
        gather_dma(c, 0).wait()
        out_copy(c, 0)
        gather_dma(c + 1, 1).wait()
        out_copy(c + 1, 1)

    return gather_kernel


BJ = 512


def _transpose_first_body(x_ref, o_ref):
    x = x_ref[...]
    xr = x.reshape(BJ, LPR, 128)
    y = jnp.transpose(xr, (1, 0, 2))
    z = jnp.transpose(y, (0, 2, 1))
    o_ref[0] = z.reshape(VPAD, BJ)[:VOCAB]


def _transpose_chain_body(acc_ref, x_ref, o_ref):
    del acc_ref
    _transpose_first_body(x_ref, o_ref)


def _mid_spec():
    return pl.BlockSpec((BJ * LPR, 128), lambda t, j: (t * (B // BJ) + j, 0))


def _make_transpose(g):
    out_spec = pl.BlockSpec((1, VOCAB, BJ), lambda t, j: (g * TG + t, 0, j))
    out_shape = jax.ShapeDtypeStruct((T, VOCAB, B), jnp.float32)
    if g == 0:
        return pl.pallas_call(
            _transpose_first_body,
            grid=(TG, B // BJ),
            in_specs=[_mid_spec()],
            out_specs=out_spec,
            out_shape=out_shape,
        )
    return pl.pallas_call(
        _transpose_chain_body,
        grid=(TG, B // BJ),
        in_specs=[pl.BlockSpec(memory_space=pltpu.MemorySpace.HBM), _mid_spec()],
        out_specs=out_spec,
        out_shape=out_shape,
        input_output_aliases={0: 0},
    )


_gathers = [_make_gather(g) for g in range(NG)]
_transposes = [_make_transpose(g) for g in range(NG)]


@jax.jit
def kernel(idx, token_embedding_table):
    idx_t = jnp.transpose(idx)
    idx8 = (idx_t.reshape(-1)[:, None] * LPR
            + jnp.arange(LPR, dtype=jnp.int32)).reshape(-1)
    table_p = jnp.pad(token_embedding_table, ((0, 0), (0, VPAD - VOCAB)))
    tview = table_p.reshape(VOCAB * LPR, 128)
    mids = [_gathers[g](tview, idx8) for g in range(NG)]
    out_t = _transposes[0](mids[0])
    for g in range(1, NG):
        out_t = _transposes[g](out_t, mids[g])
    return jnp.transpose(out_t, (2, 0, 1))
